# baseline (speedup 1.0000x reference)
import functools

import jax
from jax import lax
import jax.numpy as jnp
from jax.experimental import pallas as pl
from jax.experimental.pallas import tpu as pltpu
from jax.experimental.pallas import tpu_sc as plsc


_CH = 1024
_NBUF = 3
_NBUF_OUT = 2


def _make_body(B, S, D, ch, nbuf, nbuf_out):
    st = S // ch
    T = st * B

    def body(x_hbm, pos_hbm, o_hbm, xbuf, posbuf, obuf, in_sems, pos_sems,
             out_sems):
        def chunk(t):
            return divmod(t, B)

        def make_in(t):
            s, b = chunk(t)
            return pltpu.make_async_copy(
                x_hbm.at[b, pl.ds(s * ch, ch), :], xbuf.at[t % nbuf],
                in_sems.at[t % nbuf])

        def make_pos(s):
            return pltpu.make_async_copy(
                pos_hbm.at[pl.ds(s * ch, ch), :], posbuf.at[s % 2],
                pos_sems.at[s % 2])

        def make_out(t):
            s, b = chunk(t)
            return pltpu.make_async_copy(
                obuf.at[t % nbuf_out], o_hbm.at[b, pl.ds(s * ch, ch), :],
                out_sems.at[t % nbuf_out])

        in_copies, out_copies, pos_copies = {}, {}, {}
        pos_copies[0] = make_pos(0)
        pos_copies[0].start()
        for t in range(min(nbuf, T)):
            in_copies[t] = make_in(t)
            in_copies[t].start()
        for t in range(T):
            s, b = chunk(t)
            if b == 0:
                pos_copies[s].wait()
                if s + 1 < st:
                    pos_copies[s + 1] = make_pos(s + 1)
                    pos_copies[s + 1].start()
            in_copies[t].wait()
            if t >= nbuf_out:
                out_copies[t - nbuf_out].wait()
            obuf[t % nbuf_out] = xbuf[t % nbuf] + posbuf[s % 2]
            out_copies[t] = make_out(t)
            out_copies[t].start()
            if t + nbuf < T:
                in_copies[t + nbuf] = make_in(t + nbuf)
                in_copies[t + nbuf].start()
        for t in range(max(0, T - nbuf_out), T):
            out_copies[t].wait()

    return body


def _tc_kernel(x, pos_table):
    B, S, D = x.shape
    ch = _CH if S % _CH == 0 else S
    nbuf = min(_NBUF, (S // ch) * B)
    nbuf_out = min(_NBUF_OUT, (S // ch) * B)
    out = pl.pallas_call(
        _make_body(B, S, D, ch, nbuf, nbuf_out),
        in_specs=[
            pl.BlockSpec(memory_space=pltpu.HBM),
            pl.BlockSpec(memory_space=pltpu.HBM),
        ],
        out_specs=pl.BlockSpec(memory_space=pltpu.HBM),
        out_shape=jax.ShapeDtypeStruct((B, S, D), x.dtype),
        scratch_shapes=[
            pltpu.VMEM((nbuf, ch, D), x.dtype),
            pltpu.VMEM((2, ch, D), x.dtype),
            pltpu.VMEM((nbuf_out, ch, D), x.dtype),
            pltpu.SemaphoreType.DMA((nbuf,)),
            pltpu.SemaphoreType.DMA((2,)),
            pltpu.SemaphoreType.DMA((nbuf_out,)),
        ],
    )(x, pos_table)
    return out


_SC_CH = 4
_SC_NSLOT = 3


def _sc_kernel(x, pos_table, nw, nc):
    B, S, D = x.shape
    rows_w = S // nw
    seg = _SC_CH * D
    nchunks = rows_w // _SC_CH
    nslot = _SC_NSLOT
    nvec = seg // 16
    unroll = 8

    xf = x.reshape(B, S * D)
    pf = pos_table.reshape(-1)
    mesh = plsc.VectorSubcoreMesh(core_axis_name="c", subcore_axis_name="s")

    scratch = (
        [pltpu.VMEM((seg,), x.dtype)] * (nslot * B)
        + [pltpu.VMEM((seg,), x.dtype)] * nslot
        + [pltpu.SemaphoreType.DMA] * (2 * nslot)
    )

    @functools.partial(
        pl.kernel,
        out_type=jax.ShapeDtypeStruct((B, S * D), x.dtype),
        mesh=mesh,
        scratch_types=scratch,
    )
    def sck(x_hbm, pos_hbm, o_hbm, *scr):
        xbufs = [[scr[sl * B + b] for b in range(B)] for sl in range(nslot)]
        posbufs = list(scr[nslot * B:nslot * B + nslot])
        in_sems = list(scr[nslot * B + nslot:nslot * B + 2 * nslot])
        out_sems = list(scr[nslot * B + 2 * nslot:])
        wid = lax.axis_index("s") * nc + lax.axis_index("c")
        base = wid * (rows_w * D)

        def make_in(c):
            sl = c % nslot
            cps = [pltpu.make_async_copy(
                pos_hbm.at[pl.ds(base + c * seg, seg)], posbufs[sl],
                in_sems[sl])]
            for b in range(B):
                cps.append(pltpu.make_async_copy(
                    x_hbm.at[b, pl.ds(base + c * seg, seg)], xbufs[sl][b],
                    in_sems[sl]))
            return cps

        def make_out(c):
            sl = c % nslot
            return [pltpu.make_async_copy(
                xbufs[sl][b], o_hbm.at[b, pl.ds(base + c * seg, seg)],
                out_sems[sl]) for b in range(B)]

        def compute(c):
            sl = c % nslot

            @plsc.parallel_loop(0, nvec * 16, step=16, unroll=unroll)
            def _(off):
                vp = posbufs[sl][pl.ds(off, 16)]
                for b in range(B):
                    xbufs[sl][b][pl.ds(off, 16)] = (
                        xbufs[sl][b][pl.ds(off, 16)] + vp)

        in_copies, out_copies = {}, {}
        for c in range(min(2, nchunks)):
            in_copies[c] = make_in(c)
            for cp in in_copies[c]:
                cp.start()
        for c in range(nchunks):
            for cp in in_copies[c]:
                cp.wait()
            compute(c)
            out_copies[c] = make_out(c)
            for cp in out_copies[c]:
                cp.start()
            if c >= 1:
                for cp in out_copies[c - 1]:
                    cp.wait()
            if c + 2 < nchunks:
                in_copies[c + 2] = make_in(c + 2)
                for cp in in_copies[c + 2]:
                    cp.start()
        for c in range(max(0, nchunks - 1), nchunks):
            for cp in out_copies[c]:
                cp.wait()

    out = sck(xf, pf)
    return out.reshape(B, S, D)


def kernel(x, pos_table):
    B, S, D = x.shape
    try:
        info = plsc.get_sparse_core_info()
        nw = info.num_cores * info.num_subcores
        nc = info.num_cores
    except Exception:
        nw = 0
        nc = 0
    sc_ok = (
        nw > 0
        and x.dtype == jnp.float32
        and pos_table.dtype == jnp.float32
        and D % 64 == 0
        and S % (nw * _SC_CH) == 0
        and (_SC_CH * D) % 8 == 0
    )
    if sc_ok:
        return _sc_kernel(x, pos_table, nw, nc)
    return _tc_kernel(x, pos_table)


# speedup vs baseline: 4.0709x; 4.0709x over previous
import jax
import jax.numpy as jnp
from jax.experimental import pallas as pl
from jax.experimental.pallas import tpu as pltpu

_CH = 1024
_NBUF = 3
_NBUF_OUT = 2


def _make_body(B, S, D, ch, nbuf, nbuf_out):
    st = S // ch
    T = st * B

    def body(x_hbm, pos_hbm, o_hbm, xbuf, posbuf, obuf, in_sems, pos_sems,
             out_sems):
        def chunk(t):
            return divmod(t, B)

        def make_in(t):
            s, b = chunk(t)
            return pltpu.make_async_copy(
                x_hbm.at[b, pl.ds(s * ch, ch), :], xbuf.at[t % nbuf],
                in_sems.at[t % nbuf])

        def make_pos(s):
            return pltpu.make_async_copy(
                pos_hbm.at[pl.ds(s * ch, ch), :], posbuf.at[s % 2],
                pos_sems.at[s % 2])

        def make_out(t):
            s, b = chunk(t)
            return pltpu.make_async_copy(
                obuf.at[t % nbuf_out], o_hbm.at[b, pl.ds(s * ch, ch), :],
                out_sems.at[t % nbuf_out])

        in_copies, out_copies, pos_copies = {}, {}, {}
        pos_copies[0] = make_pos(0)
        pos_copies[0].start()
        for t in range(min(nbuf, T)):
            in_copies[t] = make_in(t)
            in_copies[t].start()
        for t in range(T):
            s, b = chunk(t)
            if b == 0:
                pos_copies[s].wait()
                if s + 1 < st:
                    pos_copies[s + 1] = make_pos(s + 1)
                    pos_copies[s + 1].start()
            in_copies[t].wait()
            if t >= nbuf_out:
                out_copies[t - nbuf_out].wait()
            obuf[t % nbuf_out] = xbuf[t % nbuf] + posbuf[s % 2]
            out_copies[t] = make_out(t)
            out_copies[t].start()
            if t + nbuf < T:
                in_copies[t + nbuf] = make_in(t + nbuf)
                in_copies[t + nbuf].start()
        for t in range(max(0, T - nbuf_out), T):
            out_copies[t].wait()

    return body


def kernel(x, pos_table):
    B, S, D = x.shape
    ch = _CH if S % _CH == 0 else S
    nbuf = min(_NBUF, (S // ch) * B)
    nbuf_out = min(_NBUF_OUT, (S // ch) * B)
    out = pl.pallas_call(
        _make_body(B, S, D, ch, nbuf, nbuf_out),
        in_specs=[
            pl.BlockSpec(memory_space=pltpu.HBM),
            pl.BlockSpec(memory_space=pltpu.HBM),
        ],
        out_specs=pl.BlockSpec(memory_space=pltpu.HBM),
        out_shape=jax.ShapeDtypeStruct((B, S, D), x.dtype),
        scratch_shapes=[
            pltpu.VMEM((nbuf, ch, D), x.dtype),
            pltpu.VMEM((2, ch, D), x.dtype),
            pltpu.VMEM((nbuf_out, ch, D), x.dtype),
            pltpu.SemaphoreType.DMA((nbuf,)),
            pltpu.SemaphoreType.DMA((2,)),
            pltpu.SemaphoreType.DMA((nbuf_out,)),
        ],
    )(x, pos_table)
    return out
